# Initial kernel scaffold; baseline (speedup 1.0000x reference)
import jax
import jax.numpy as jnp
from jax.experimental import pallas as pl


def kernel(lr_data, hr_data, lr_doy, hr_doy, target_doy):
    raise NotImplementedError("write your pallas kernel here")


# speedup vs baseline: 2.4045x; 2.4045x over previous
import numpy as np
import jax
import jax.numpy as jnp
from jax import lax
from jax.experimental import pallas as pl
from jax.experimental.pallas import tpu as pltpu


def _resize_matrix(in_size: int, out_size: int) -> np.ndarray:
    sample_f = (np.arange(out_size) + 0.5) * (in_size / out_size) - 0.5
    x = np.abs(sample_f[:, None] - np.arange(in_size)[None, :])
    w = np.maximum(0.0, 1.0 - x)
    w = w / w.sum(axis=1, keepdims=True)
    return w.astype(np.float32)


def _index_kernel(lr_doy_ref, hr_doy_ref, tgt_ref,
                  lri0_ref, lri1_ref, hri0_ref, hri1_ref):
    t = tgt_ref[...].astype(jnp.float32)

    def one(doy_ref, i0_ref, i1_ref):
        d = doy_ref[...].astype(jnp.float32)
        T = d.shape[1]
        cmp = (d[:, :, None] < t[0][None, None, :]).astype(jnp.int32)
        idx = jnp.sum(cmp, axis=1)
        i1 = jnp.clip(idx, 1, T - 1)
        i0_ref[...] = i1 - 1
        i1_ref[...] = i1

    one(lr_doy_ref, lri0_ref, lri1_ref)
    one(hr_doy_ref, hri0_ref, hri1_ref)


def _fuse_kernel(lri0_p, lri1_p, hri0_p, hri1_p, lr_doy_p, hr_doy_p, tgt_p,
                 lr0_ref, lr1_ref, hr0_ref, hr1_ref, m_ref,
                 out_lr_ref, out_hr_ref):
    b = pl.program_id(0)
    t = pl.program_id(1)
    tf = tgt_p[t].astype(jnp.float32)

    def weight(doy_p, i0_p, i1_p):
        d0 = doy_p[b, i0_p[b, t]].astype(jnp.float32)
        d1 = doy_p[b, i1_p[b, t]].astype(jnp.float32)
        denom = jnp.where(d1 - d0 == 0.0, 1.0, d1 - d0)
        return jnp.clip((tf - d0) / denom, 0.0, 1.0)

    wl = weight(lr_doy_p, lri0_p, lri1_p)
    wh = weight(hr_doy_p, hri0_p, hri1_p)

    hr0 = hr0_ref[0, 0]
    hr1 = hr1_ref[0, 0]
    out_hr_ref[0, 0] = hr0 * (1.0 - wh) + hr1 * wh

    lr = lr0_ref[0, 0] * (1.0 - wl) + lr1_ref[0, 0] * wl
    m = m_ref[...]
    a = lax.dot_general(lr, m, (((1,), (1,)), ((), ())),
                        preferred_element_type=jnp.float32)
    out = lax.dot_general(a, m, (((1,), (1,)), ((), ())),
                          preferred_element_type=jnp.float32)
    out_lr_ref[0, 0] = out


def kernel(lr_data, hr_data, lr_doy, hr_doy, target_doy):
    B, Tl, C, H, W = lr_data.shape
    _, Th, _, Hh, Wh = hr_data.shape
    Tt = target_doy.shape[0]
    Hout, Wout = Hh, Wh

    tgt2d = target_doy.reshape(1, Tt)

    lri0, lri1, hri0, hri1 = pl.pallas_call(
        _index_kernel,
        out_shape=[jax.ShapeDtypeStruct((B, Tt), jnp.int32)] * 4,
    )(lr_doy, hr_doy, tgt2d)

    m = jnp.asarray(_resize_matrix(H, Hout))

    grid_spec = pltpu.PrefetchScalarGridSpec(
        num_scalar_prefetch=7,
        grid=(B, Tt),
        in_specs=[
            pl.BlockSpec((1, 1, C, H, W),
                         lambda b, t, i0, i1, j0, j1, *_: (b, i0[b, t], 0, 0, 0)),
            pl.BlockSpec((1, 1, C, H, W),
                         lambda b, t, i0, i1, j0, j1, *_: (b, i1[b, t], 0, 0, 0)),
            pl.BlockSpec((1, 1, C, Hh, Wh),
                         lambda b, t, i0, i1, j0, j1, *_: (b, j0[b, t], 0, 0, 0)),
            pl.BlockSpec((1, 1, C, Hh, Wh),
                         lambda b, t, i0, i1, j0, j1, *_: (b, j1[b, t], 0, 0, 0)),
            pl.BlockSpec((Hout, H), lambda *_: (0, 0)),
        ],
        out_specs=[
            pl.BlockSpec((1, 1, C, Hout, Wout), lambda b, t, *_: (b, t, 0, 0, 0)),
            pl.BlockSpec((1, 1, C, Hh, Wh), lambda b, t, *_: (b, t, 0, 0, 0)),
        ],
    )
    out_lr, out_hr = pl.pallas_call(
        _fuse_kernel,
        grid_spec=grid_spec,
        out_shape=[
            jax.ShapeDtypeStruct((B, Tt, C, Hout, Wout), jnp.float32),
            jax.ShapeDtypeStruct((B, Tt, C, Hh, Wh), jnp.float32),
        ],
    )(lri0, lri1, hri0, hri1, lr_doy, hr_doy, target_doy,
      lr_data, lr_data, hr_data, hr_data, m)

    return (out_lr, out_hr)
